# baseline (speedup 1.0000x reference)
import functools

import jax
import jax.numpy as jnp
from jax import lax
from jax.experimental import pallas as pl
from jax.experimental.pallas import tpu as pltpu
from jax.experimental.pallas import tpu_sc as plsc

N = 10000
E = 320000
D = 128
EPS = 1e-5

NC = 2
NS = 16
NW = NC * NS
EPW = E // NW
CHUNK = 80
SB = 2
NCH_SB = 63
SB_LEN = NCH_SB * CHUNK
EPW_PAD = SB * SB_LEN
RPT = 632
N_PAD = NS * RPT
CW = 128


def _mesh():
    return plsc.VectorSubcoreMesh(core_axis_name="c", subcore_axis_name="s",
                                  num_cores=NC, num_subcores=NS)


def _sc_agg_body(with_count, *refs):
    if with_count:
        (table, packed2, zrow, ones_h, out, cnt_out, packed_all,
         sa_a, da_a, sa_b, da_b, sa_c, da_c, rows_a, rows_b, rows_c, acc,
         sem_a, sem_b, sem_c) = refs
    else:
        (table, packed2, zrow, out, packed_all,
         sa_a, da_a, sa_b, da_b, sa_c, da_c, rows_a, rows_b, rows_c, acc,
         sem_a, sem_b, sem_c) = refs

    bufs = ((sa_a, da_a, rows_a, sem_a),
            (sa_b, da_b, rows_b, sem_b),
            (sa_c, da_c, rows_c, sem_c))

    c = lax.axis_index("c")
    s = lax.axis_index("s")
    wid = s * NC + c

    r0 = s * RPT
    pltpu.sync_copy(zrow.at[pl.ds(r0, RPT)], acc.at[pl.ds(r0, RPT)])

    def stage(sb):
        pltpu.sync_copy(packed2.at[wid, sb], packed_all)

    def unpack(i, sa, da):
        for k in range(CHUNK // 16):
            w = packed_all[i, pl.ds(k * 16, 16)]
            sa[pl.ds(k * 16, 16)] = w & 0xFFFF
            da[pl.ds(k * 16, 16)] = jnp.right_shift(w, 16)

    def gather(sa, buf, sem):
        pltpu.async_copy(table.at[sa], buf, sem)

    def wait_gather(sa, buf, sem):
        pltpu.make_async_copy(table.at[sa], buf, sem).wait()

    if with_count:
        pltpu.sync_copy(ones_h, rows_a)
        plsc.subcore_barrier()

        def cstep(i, carry):
            unpack(i, sa_a, da_a)
            pltpu.sync_copy(rows_a, acc.at[da_a], add=True)
            return carry

        for sb in range(SB):
            stage(sb)
            lax.fori_loop(0, NCH_SB, cstep, 0)
        plsc.subcore_barrier()
        pltpu.sync_copy(acc.at[pl.ds(r0, RPT)], cnt_out.at[c, pl.ds(r0, RPT)])
        pltpu.sync_copy(zrow.at[pl.ds(r0, RPT)], acc.at[pl.ds(r0, RPT)])

    plsc.subcore_barrier()

    for sb in range(SB):
        stage(sb)
        unpack(0, sa_a, da_a)
        gather(sa_a, rows_a, sem_a)
        unpack(1, sa_b, da_b)
        gather(sa_b, rows_b, sem_b)

        def trip(g, carry):
            c0 = 3 * g
            for u in range(3):
                sau, dau, rowsu, semu = bufs[u]
                saw, daw, rowsw, semw = bufs[(u + 2) % 3]
                wait_gather(sau, rowsu, semu)
                pltpu.sync_copy(rowsu, acc.at[dau], add=True)
                unpack(c0 + u + 2, saw, daw)
                gather(saw, rowsw, semw)
            return carry

        lax.fori_loop(0, (NCH_SB - 3) // 3, trip, 0)
        wait_gather(sa_a, rows_a, sem_a)
        pltpu.sync_copy(rows_a, acc.at[da_a], add=True)
        unpack(NCH_SB - 1, sa_c, da_c)
        gather(sa_c, rows_c, sem_c)
        wait_gather(sa_b, rows_b, sem_b)
        pltpu.sync_copy(rows_b, acc.at[da_b], add=True)
        wait_gather(sa_c, rows_c, sem_c)
        pltpu.sync_copy(rows_c, acc.at[da_c], add=True)

    plsc.subcore_barrier()

    pltpu.sync_copy(acc.at[pl.ds(r0, RPT)], out.at[c, pl.ds(r0, RPT)])


@functools.lru_cache(maxsize=None)
def _sc_agg(with_count):
    if with_count:
        out_type = (jax.ShapeDtypeStruct((NC, N_PAD, D), jnp.float32),
                    jax.ShapeDtypeStruct((NC, N_PAD, CW), jnp.float32))
    else:
        out_type = jax.ShapeDtypeStruct((NC, N_PAD, D), jnp.float32)
    return pl.kernel(
        functools.partial(_sc_agg_body, with_count),
        out_type=out_type,
        mesh=_mesh(),
        scratch_types=(
            pltpu.VMEM((NCH_SB, CHUNK), jnp.int32),
            pltpu.VMEM((CHUNK,), jnp.int32),
            pltpu.VMEM((CHUNK,), jnp.int32),
            pltpu.VMEM((CHUNK,), jnp.int32),
            pltpu.VMEM((CHUNK,), jnp.int32),
            pltpu.VMEM((CHUNK,), jnp.int32),
            pltpu.VMEM((CHUNK,), jnp.int32),
            pltpu.VMEM((CHUNK, D), jnp.float32),
            pltpu.VMEM((CHUNK, D), jnp.float32),
            pltpu.VMEM((CHUNK, D), jnp.float32),
            pltpu.VMEM_SHARED((N_PAD, D), jnp.float32),
            pltpu.SemaphoreType.DMA,
            pltpu.SemaphoreType.DMA,
            pltpu.SemaphoreType.DMA,
        ))


def _relu_body(x_ref, o_ref):
    o_ref[...] = jnp.maximum(x_ref[...], 0.0)


def _tc_relu(x):
    return pl.pallas_call(
        _relu_body,
        out_shape=jax.ShapeDtypeStruct((N, D), jnp.float32),
    )(x)


def _tc_layer_body(p_ref, cnt_ref, xin_ref, wl_ref, bl_ref, wr_ref,
                   g_ref, b_ref, out_ref):
    ssum = p_ref[0, :N] + p_ref[1, :N]
    cn = cnt_ref[0, :N] + cnt_ref[1, :N]
    inv = 1.0 / jnp.maximum(cn[:, 0:1], 1.0)
    agg = ssum * inv
    t = (jnp.dot(agg, wl_ref[...], preferred_element_type=jnp.float32)
         + jnp.dot(xin_ref[...], wr_ref[...], preferred_element_type=jnp.float32)
         + bl_ref[...][None, :])
    mean = jnp.mean(t, axis=0)
    var = jnp.mean((t - mean[None, :]) ** 2, axis=0)
    h = (g_ref[...][None, :] * (t - mean[None, :]) * lax.rsqrt(var + EPS)
         + b_ref[...][None, :])
    out_ref[...] = jnp.maximum(h, 0.0)


def _tc_layer(p, cnt, xin, wl, bl, wr, g, b):
    return pl.pallas_call(
        _tc_layer_body,
        out_shape=jax.ShapeDtypeStruct((N, D), jnp.float32),
    )(p, cnt, xin, wl, bl, wr, g, b)


def kernel(x, edge_index, Wl1, bl1, Wr1, g1, b1, Wl2, bl2, Wr2, g2, b2):
    src = edge_index[0].astype(jnp.int32)
    dst = edge_index[1].astype(jnp.int32)
    packed = (src | (dst << 16)).reshape(NW, EPW)
    fill = jnp.full((NW, EPW_PAD - EPW), (N_PAD - 1) << 16, jnp.int32)
    packed2 = jnp.concatenate([packed, fill], axis=1).reshape(
        NW, SB, NCH_SB, CHUNK)
    zrow = jnp.zeros((N_PAD, D), jnp.float32)
    ones_h = jnp.ones((CHUNK, CW), jnp.float32)

    r1 = _tc_relu(x)
    p1, cnt = _sc_agg(True)(r1, packed2, zrow, ones_h)
    h1 = _tc_layer(p1, cnt, x, Wl1, bl1, Wr1, g1, b1)
    p2 = _sc_agg(False)(h1, packed2, zrow)
    out = _tc_layer(p2, cnt, h1, Wl2, bl2, Wr2, g2, b2)
    return out


# speedup vs baseline: 1.0766x; 1.0766x over previous
import functools

import jax
import jax.numpy as jnp
from jax import lax
from jax.experimental import pallas as pl
from jax.experimental.pallas import tpu as pltpu
from jax.experimental.pallas import tpu_sc as plsc

N = 10000
E = 320000
D = 128
EPS = 1e-5

NC = 2
NS = 16
NW = NC * NS
EPW = E // NW
CHUNK = 80
SB = 2
NCH_SB = 63
SB_LEN = NCH_SB * CHUNK
EPW_PAD = SB * SB_LEN
RPT = 632
N_PAD = NS * RPT
CW = 128


def _mesh():
    return plsc.VectorSubcoreMesh(core_axis_name="c", subcore_axis_name="s",
                                  num_cores=NC, num_subcores=NS)


def _sc_agg_body(with_count, *refs):
    if with_count:
        (table, packed2, zrow, ones_h, out, cnt_out, packed_all,
         sa_a, da_a, sa_b, da_b, sa_c, da_c, rows_a, rows_b, rows_c, acc,
         sem_a, sem_b, sem_c) = refs
    else:
        (table, packed2, zrow, out, packed_all,
         sa_a, da_a, sa_b, da_b, sa_c, da_c, rows_a, rows_b, rows_c, acc,
         sem_a, sem_b, sem_c) = refs

    bufs = ((sa_a, da_a, rows_a, sem_a),
            (sa_b, da_b, rows_b, sem_b),
            (sa_c, da_c, rows_c, sem_c))

    c = lax.axis_index("c")
    s = lax.axis_index("s")
    wid = s * NC + c

    r0 = s * RPT
    pltpu.sync_copy(zrow.at[pl.ds(r0, RPT)], acc.at[pl.ds(r0, RPT)])

    def stage(sb):
        pltpu.sync_copy(packed2.at[wid, sb], packed_all)

    def unpack(i, sa, da):
        for k in range(CHUNK // 16):
            w = packed_all[i, pl.ds(k * 16, 16)]
            sa[pl.ds(k * 16, 16)] = w & 0xFFFF
            da[pl.ds(k * 16, 16)] = jnp.right_shift(w, 16)

    def gather(sa, buf, sem):
        pltpu.async_copy(table.at[sa], buf, sem)

    def wait_gather(sa, buf, sem):
        pltpu.make_async_copy(table.at[sa], buf, sem).wait()

    if with_count:
        pltpu.sync_copy(ones_h, rows_a)
        plsc.subcore_barrier()

        def cstep(i, carry):
            unpack(i, sa_a, da_a)
            pltpu.sync_copy(rows_a, acc.at[da_a], add=True)
            return carry

        for sb in range(SB):
            stage(sb)
            lax.fori_loop(0, NCH_SB, cstep, 0)
        plsc.subcore_barrier()
        pltpu.sync_copy(acc.at[pl.ds(r0, RPT)], cnt_out.at[c, pl.ds(r0, RPT)])
        pltpu.sync_copy(zrow.at[pl.ds(r0, RPT)], acc.at[pl.ds(r0, RPT)])

    plsc.subcore_barrier()

    for sb in range(SB):
        stage(sb)
        for u in range(3):
            sau, dau, rowsu, semu = bufs[u]
            unpack(u, sau, dau)
            gather(sau, rowsu, semu)

        def trip(g, carry):
            c0 = 3 * g
            for u in range(3):
                sau, dau, rowsu, semu = bufs[u]
                wait_gather(sau, rowsu, semu)
                pltpu.sync_copy(rowsu, acc.at[dau], add=True)
                unpack(c0 + u + 3, sau, dau)
                gather(sau, rowsu, semu)
            return carry

        lax.fori_loop(0, (NCH_SB - 3) // 3, trip, 0)
        for u in range(3):
            sau, dau, rowsu, semu = bufs[u]
            wait_gather(sau, rowsu, semu)
            pltpu.sync_copy(rowsu, acc.at[dau], add=True)

    plsc.subcore_barrier()

    pltpu.sync_copy(acc.at[pl.ds(r0, RPT)], out.at[c, pl.ds(r0, RPT)])


@functools.lru_cache(maxsize=None)
def _sc_agg(with_count):
    if with_count:
        out_type = (jax.ShapeDtypeStruct((NC, N_PAD, D), jnp.float32),
                    jax.ShapeDtypeStruct((NC, N_PAD, CW), jnp.float32))
    else:
        out_type = jax.ShapeDtypeStruct((NC, N_PAD, D), jnp.float32)
    return pl.kernel(
        functools.partial(_sc_agg_body, with_count),
        out_type=out_type,
        mesh=_mesh(),
        scratch_types=(
            pltpu.VMEM((NCH_SB, CHUNK), jnp.int32),
            pltpu.VMEM((CHUNK,), jnp.int32),
            pltpu.VMEM((CHUNK,), jnp.int32),
            pltpu.VMEM((CHUNK,), jnp.int32),
            pltpu.VMEM((CHUNK,), jnp.int32),
            pltpu.VMEM((CHUNK,), jnp.int32),
            pltpu.VMEM((CHUNK,), jnp.int32),
            pltpu.VMEM((CHUNK, D), jnp.float32),
            pltpu.VMEM((CHUNK, D), jnp.float32),
            pltpu.VMEM((CHUNK, D), jnp.float32),
            pltpu.VMEM_SHARED((N_PAD, D), jnp.float32),
            pltpu.SemaphoreType.DMA,
            pltpu.SemaphoreType.DMA,
            pltpu.SemaphoreType.DMA,
        ))


def _relu_body(x_ref, o_ref):
    o_ref[...] = jnp.maximum(x_ref[...], 0.0)


def _tc_relu(x):
    return pl.pallas_call(
        _relu_body,
        out_shape=jax.ShapeDtypeStruct((N, D), jnp.float32),
    )(x)


def _tc_layer_body(p_ref, cnt_ref, xin_ref, wl_ref, bl_ref, wr_ref,
                   g_ref, b_ref, out_ref):
    ssum = p_ref[0, :N] + p_ref[1, :N]
    cn = cnt_ref[0, :N] + cnt_ref[1, :N]
    inv = 1.0 / jnp.maximum(cn[:, 0:1], 1.0)
    agg = ssum * inv
    t = (jnp.dot(agg, wl_ref[...], preferred_element_type=jnp.float32)
         + jnp.dot(xin_ref[...], wr_ref[...], preferred_element_type=jnp.float32)
         + bl_ref[...][None, :])
    mean = jnp.mean(t, axis=0)
    var = jnp.mean((t - mean[None, :]) ** 2, axis=0)
    h = (g_ref[...][None, :] * (t - mean[None, :]) * lax.rsqrt(var + EPS)
         + b_ref[...][None, :])
    out_ref[...] = jnp.maximum(h, 0.0)


def _tc_layer(p, cnt, xin, wl, bl, wr, g, b):
    return pl.pallas_call(
        _tc_layer_body,
        out_shape=jax.ShapeDtypeStruct((N, D), jnp.float32),
    )(p, cnt, xin, wl, bl, wr, g, b)


def kernel(x, edge_index, Wl1, bl1, Wr1, g1, b1, Wl2, bl2, Wr2, g2, b2):
    src = edge_index[0].astype(jnp.int32)
    dst = edge_index[1].astype(jnp.int32)
    packed = (src | (dst << 16)).reshape(NW, EPW)
    fill = jnp.full((NW, EPW_PAD - EPW), (N_PAD - 1) << 16, jnp.int32)
    packed2 = jnp.concatenate([packed, fill], axis=1).reshape(
        NW, SB, NCH_SB, CHUNK)
    zrow = jnp.zeros((N_PAD, D), jnp.float32)
    ones_h = jnp.ones((CHUNK, CW), jnp.float32)

    r1 = _tc_relu(x)
    p1, cnt = _sc_agg(True)(r1, packed2, zrow, ones_h)
    h1 = _tc_layer(p1, cnt, x, Wl1, bl1, Wr1, g1, b1)
    p2 = _sc_agg(False)(h1, packed2, zrow)
    out = _tc_layer(p2, cnt, h1, Wl2, bl2, Wr2, g2, b2)
    return out


# speedup vs baseline: 1.0783x; 1.0016x over previous
import functools

import jax
import jax.numpy as jnp
from jax import lax
from jax.experimental import pallas as pl
from jax.experimental.pallas import tpu as pltpu
from jax.experimental.pallas import tpu_sc as plsc

N = 10000
E = 320000
D = 128
EPS = 1e-5

NC = 2
NS = 16
NW = NC * NS
EPW = E // NW
CHUNK = 80
SB = 2
NCH_SB = 63
SB_LEN = NCH_SB * CHUNK
EPW_PAD = SB * SB_LEN
RPT = 632
N_PAD = NS * RPT
CW = 128


def _mesh():
    return plsc.VectorSubcoreMesh(core_axis_name="c", subcore_axis_name="s",
                                  num_cores=NC, num_subcores=NS)


def _sc_agg_body(with_count, *refs):
    if with_count:
        (table, packed2, zrow, ones_h, out, cnt_out, packed_all,
         sa_a, da_a, sa_b, da_b, sa_c, da_c, rows_a, rows_b, rows_c, acc,
         sem_a, sem_b, sem_c) = refs
    else:
        (table, packed2, zrow, out, packed_all,
         sa_a, da_a, sa_b, da_b, sa_c, da_c, rows_a, rows_b, rows_c, acc,
         sem_a, sem_b, sem_c) = refs

    bufs = ((sa_a, da_a, rows_a, sem_a),
            (sa_b, da_b, rows_b, sem_b),
            (sa_c, da_c, rows_c, sem_c))

    c = lax.axis_index("c")
    s = lax.axis_index("s")
    wid = s * NC + c

    r0 = s * RPT
    pltpu.sync_copy(zrow.at[pl.ds(r0, RPT)], acc.at[pl.ds(r0, RPT)])

    def stage(sb):
        pltpu.sync_copy(packed2.at[wid, sb], packed_all)

    def unpack(i, sa, da):
        for k in range(CHUNK // 16):
            w = packed_all[i, pl.ds(k * 16, 16)]
            sa[pl.ds(k * 16, 16)] = w & 0xFFFF
            da[pl.ds(k * 16, 16)] = jnp.right_shift(w, 16)

    def gather(sa, buf, sem):
        pltpu.async_copy(table.at[sa], buf, sem)

    def wait_gather(sa, buf, sem):
        pltpu.make_async_copy(table.at[sa], buf, sem).wait()

    if with_count:
        pltpu.sync_copy(ones_h, rows_a)
        plsc.subcore_barrier()

        def cstep(i, carry):
            unpack(i, sa_a, da_a)
            pltpu.sync_copy(rows_a, acc.at[da_a], add=True)
            return carry

        for sb in range(SB):
            stage(sb)
            lax.fori_loop(0, NCH_SB, cstep, 0)
        plsc.subcore_barrier()
        pltpu.sync_copy(acc.at[pl.ds(r0, RPT)], cnt_out.at[c, pl.ds(r0, RPT)])
        pltpu.sync_copy(zrow.at[pl.ds(r0, RPT)], acc.at[pl.ds(r0, RPT)])

    plsc.subcore_barrier()

    for sb in range(SB):
        stage(sb)
        for u in range(3):
            sau, dau, rowsu, semu = bufs[u]
            unpack(u, sau, dau)
            gather(sau, rowsu, semu)

        def trip(g, carry):
            c0 = 3 * g
            for u in range(3):
                sau, dau, rowsu, semu = bufs[u]
                wait_gather(sau, rowsu, semu)
                pltpu.sync_copy(rowsu, acc.at[dau], add=True)
                unpack(c0 + u + 3, sau, dau)
                gather(sau, rowsu, semu)
            return carry

        lax.fori_loop(0, (NCH_SB - 3) // 3, trip, 0)
        for u in range(3):
            sau, dau, rowsu, semu = bufs[u]
            wait_gather(sau, rowsu, semu)
            pltpu.sync_copy(rowsu, acc.at[dau], add=True)

    plsc.subcore_barrier()

    pltpu.sync_copy(acc.at[pl.ds(r0, RPT)], out.at[c, pl.ds(r0, RPT)])


@functools.lru_cache(maxsize=None)
def _sc_agg(with_count):
    if with_count:
        out_type = (jax.ShapeDtypeStruct((NC, N_PAD, D), jnp.float32),
                    jax.ShapeDtypeStruct((NC, N_PAD, CW), jnp.float32))
    else:
        out_type = jax.ShapeDtypeStruct((NC, N_PAD, D), jnp.float32)
    return pl.kernel(
        functools.partial(_sc_agg_body, with_count),
        out_type=out_type,
        mesh=_mesh(),
        scratch_types=(
            pltpu.VMEM((NCH_SB, CHUNK), jnp.int32),
            pltpu.VMEM((CHUNK,), jnp.int32),
            pltpu.VMEM((CHUNK,), jnp.int32),
            pltpu.VMEM((CHUNK,), jnp.int32),
            pltpu.VMEM((CHUNK,), jnp.int32),
            pltpu.VMEM((CHUNK,), jnp.int32),
            pltpu.VMEM((CHUNK,), jnp.int32),
            pltpu.VMEM((CHUNK, D), jnp.float32),
            pltpu.VMEM((CHUNK, D), jnp.float32),
            pltpu.VMEM((CHUNK, D), jnp.float32),
            pltpu.VMEM_SHARED((N_PAD, D), jnp.float32),
            pltpu.SemaphoreType.DMA,
            pltpu.SemaphoreType.DMA,
            pltpu.SemaphoreType.DMA,
        ))


def _relu_body(x_ref, o_ref):
    o_ref[...] = jnp.maximum(x_ref[...], 0.0)


def _tc_relu(x):
    return pl.pallas_call(
        _relu_body,
        out_shape=jax.ShapeDtypeStruct((N, D), jnp.float32),
    )(x)


def _tc_layer_body(p_ref, cnt_ref, xin_ref, wl_ref, bl_ref, wr_ref,
                   g_ref, b_ref, out_ref):
    ssum = p_ref[0, :N] + p_ref[1, :N]
    cn = cnt_ref[0, :N] + cnt_ref[1, :N]
    inv = 1.0 / jnp.maximum(cn[:, 0:1], 1.0)
    agg = ssum * inv
    t = (jnp.dot(agg, wl_ref[...], preferred_element_type=jnp.float32)
         + jnp.dot(xin_ref[...], wr_ref[...], preferred_element_type=jnp.float32)
         + bl_ref[...][None, :])
    mean = jnp.mean(t, axis=0)
    var = jnp.mean((t - mean[None, :]) ** 2, axis=0)
    h = (g_ref[...][None, :] * (t - mean[None, :]) * lax.rsqrt(var + EPS)
         + b_ref[...][None, :])
    out_ref[...] = jnp.maximum(h, 0.0)


def _tc_layer(p, cnt, xin, wl, bl, wr, g, b):
    return pl.pallas_call(
        _tc_layer_body,
        out_shape=jax.ShapeDtypeStruct((N, D), jnp.float32),
    )(p, cnt, xin, wl, bl, wr, g, b)


def kernel(x, edge_index, Wl1, bl1, Wr1, g1, b1, Wl2, bl2, Wr2, g2, b2):
    src = edge_index[0].astype(jnp.int32)
    dst = edge_index[1].astype(jnp.int32)
    packed = (src | (dst << 16)).reshape(NW, EPW)
    pad_dst = N + (jnp.arange(EPW_PAD - EPW, dtype=jnp.int32) % (N_PAD - N))
    fill = jnp.broadcast_to(pad_dst << 16, (NW, EPW_PAD - EPW))
    packed2 = jnp.concatenate([packed, fill], axis=1).reshape(
        NW, SB, NCH_SB, CHUNK)
    zrow = jnp.zeros((N_PAD, D), jnp.float32)
    ones_h = jnp.ones((CHUNK, CW), jnp.float32)

    r1 = _tc_relu(x)
    p1, cnt = _sc_agg(True)(r1, packed2, zrow, ones_h)
    h1 = _tc_layer(p1, cnt, x, Wl1, bl1, Wr1, g1, b1)
    p2 = _sc_agg(False)(h1, packed2, zrow)
    out = _tc_layer(p2, cnt, h1, Wl2, bl2, Wr2, g2, b2)
    return out


# speedup vs baseline: 1.2616x; 1.1699x over previous
import functools

import jax
import jax.numpy as jnp
from jax import lax
from jax.experimental import pallas as pl
from jax.experimental.pallas import tpu as pltpu
from jax.experimental.pallas import tpu_sc as plsc

N = 10000
E = 320000
D = 128
EPS = 1e-5

NC = 2
NS = 16
NW = NC * NS
EPW = E // NW
CHUNK = 64
NCH = 157
EPW_PAD = NCH * CHUNK
RPT = 632
N_PAD = NS * RPT
CW = 128


def _mesh():
    return plsc.VectorSubcoreMesh(core_axis_name="c", subcore_axis_name="s",
                                  num_cores=NC, num_subcores=NS)


def _sc_agg_body(with_count, *refs):
    if with_count:
        (table, packed2, zrow, ones_h, out, cnt_out, packed_all,
         sa_a, da_a, sa_b, da_b, sa_c, da_c, rows_a, rows_b, rows_c, acc,
         sem_a, sem_b, sem_c) = refs
    else:
        (table, packed2, zrow, out, packed_all,
         sa_a, da_a, sa_b, da_b, sa_c, da_c, rows_a, rows_b, rows_c, acc,
         sem_a, sem_b, sem_c) = refs

    bufs = ((sa_a, da_a, rows_a, sem_a),
            (sa_b, da_b, rows_b, sem_b),
            (sa_c, da_c, rows_c, sem_c))

    c = lax.axis_index("c")
    s = lax.axis_index("s")
    wid = s * NC + c

    r0 = s * RPT
    pltpu.sync_copy(zrow.at[pl.ds(r0, RPT)], acc.at[pl.ds(r0, RPT)])

    pltpu.sync_copy(packed2.at[wid], packed_all)

    def unpack(i, sa, da):
        for k in range(CHUNK // 16):
            w = packed_all[i, pl.ds(k * 16, 16)]
            sa[pl.ds(k * 16, 16)] = w & 0xFFFF
            da[pl.ds(k * 16, 16)] = jnp.right_shift(w, 16)

    def gather(sa, buf, sem):
        pltpu.async_copy(table.at[sa], buf, sem)

    def wait_gather(sa, buf, sem):
        pltpu.make_async_copy(table.at[sa], buf, sem).wait()

    if with_count:
        pltpu.sync_copy(ones_h, rows_a)
        plsc.subcore_barrier()

        def cstep(i, carry):
            unpack(i, sa_a, da_a)
            pltpu.sync_copy(rows_a, acc.at[da_a], add=True)
            return carry

        lax.fori_loop(0, NCH, cstep, 0)
        plsc.subcore_barrier()
        pltpu.sync_copy(acc.at[pl.ds(r0, RPT)], cnt_out.at[c, pl.ds(r0, RPT)])
        pltpu.sync_copy(zrow.at[pl.ds(r0, RPT)], acc.at[pl.ds(r0, RPT)])

    plsc.subcore_barrier()

    for u in range(3):
        sau, dau, rowsu, semu = bufs[u]
        unpack(u, sau, dau)
        gather(sau, rowsu, semu)

    def trip(g, carry):
        c0 = 3 * g
        for u in range(3):
            sau, dau, rowsu, semu = bufs[u]
            wait_gather(sau, rowsu, semu)
            pltpu.sync_copy(rowsu, acc.at[dau], add=True)
            unpack(c0 + u + 3, sau, dau)
            gather(sau, rowsu, semu)
        return carry

    lax.fori_loop(0, (NCH - 4) // 3, trip, 0)
    sau, dau, rowsu, semu = bufs[0]
    wait_gather(sau, rowsu, semu)
    pltpu.sync_copy(rowsu, acc.at[dau], add=True)
    unpack(NCH - 1, sau, dau)
    gather(sau, rowsu, semu)
    for u in (1, 2, 0):
        sau, dau, rowsu, semu = bufs[u]
        wait_gather(sau, rowsu, semu)
        pltpu.sync_copy(rowsu, acc.at[dau], add=True)

    plsc.subcore_barrier()

    pltpu.sync_copy(acc.at[pl.ds(r0, RPT)], out.at[c, pl.ds(r0, RPT)])


@functools.lru_cache(maxsize=None)
def _sc_agg(with_count):
    if with_count:
        out_type = (jax.ShapeDtypeStruct((NC, N_PAD, D), jnp.float32),
                    jax.ShapeDtypeStruct((NC, N_PAD, CW), jnp.float32))
    else:
        out_type = jax.ShapeDtypeStruct((NC, N_PAD, D), jnp.float32)
    return pl.kernel(
        functools.partial(_sc_agg_body, with_count),
        out_type=out_type,
        mesh=_mesh(),
        scratch_types=(
            pltpu.VMEM((NCH, CHUNK), jnp.int32),
            pltpu.VMEM((CHUNK,), jnp.int32),
            pltpu.VMEM((CHUNK,), jnp.int32),
            pltpu.VMEM((CHUNK,), jnp.int32),
            pltpu.VMEM((CHUNK,), jnp.int32),
            pltpu.VMEM((CHUNK,), jnp.int32),
            pltpu.VMEM((CHUNK,), jnp.int32),
            pltpu.VMEM((CHUNK, D), jnp.float32),
            pltpu.VMEM((CHUNK, D), jnp.float32),
            pltpu.VMEM((CHUNK, D), jnp.float32),
            pltpu.VMEM_SHARED((N_PAD, D), jnp.float32),
            pltpu.SemaphoreType.DMA,
            pltpu.SemaphoreType.DMA,
            pltpu.SemaphoreType.DMA,
        ))


def _relu_body(x_ref, o_ref):
    o_ref[...] = jnp.maximum(x_ref[...], 0.0)


def _tc_relu(x):
    return pl.pallas_call(
        _relu_body,
        out_shape=jax.ShapeDtypeStruct((N, D), jnp.float32),
    )(x)


def _tc_layer_body(p_ref, cnt_ref, xin_ref, wl_ref, bl_ref, wr_ref,
                   g_ref, b_ref, out_ref):
    ssum = p_ref[0, :N] + p_ref[1, :N]
    cn = cnt_ref[0, :N] + cnt_ref[1, :N]
    inv = 1.0 / jnp.maximum(cn[:, 0:1], 1.0)
    agg = ssum * inv
    t = (jnp.dot(agg, wl_ref[...], preferred_element_type=jnp.float32)
         + jnp.dot(xin_ref[...], wr_ref[...], preferred_element_type=jnp.float32)
         + bl_ref[...][None, :])
    mean = jnp.mean(t, axis=0)
    var = jnp.mean((t - mean[None, :]) ** 2, axis=0)
    h = (g_ref[...][None, :] * (t - mean[None, :]) * lax.rsqrt(var + EPS)
         + b_ref[...][None, :])
    out_ref[...] = jnp.maximum(h, 0.0)


def _tc_layer(p, cnt, xin, wl, bl, wr, g, b):
    return pl.pallas_call(
        _tc_layer_body,
        out_shape=jax.ShapeDtypeStruct((N, D), jnp.float32),
    )(p, cnt, xin, wl, bl, wr, g, b)


def kernel(x, edge_index, Wl1, bl1, Wr1, g1, b1, Wl2, bl2, Wr2, g2, b2):
    src = edge_index[0].astype(jnp.int32)
    dst = edge_index[1].astype(jnp.int32)
    packed = (src | (dst << 16)).reshape(NW, EPW)
    pad_dst = N + (jnp.arange(EPW_PAD - EPW, dtype=jnp.int32) % (N_PAD - N))
    fill = jnp.broadcast_to(pad_dst << 16, (NW, EPW_PAD - EPW))
    packed2 = jnp.concatenate([packed, fill], axis=1).reshape(
        NW, NCH, CHUNK)
    zrow = jnp.zeros((N_PAD, D), jnp.float32)
    ones_h = jnp.ones((CHUNK, CW), jnp.float32)

    r1 = _tc_relu(x)
    p1, cnt = _sc_agg(True)(r1, packed2, zrow, ones_h)
    h1 = _tc_layer(p1, cnt, x, Wl1, bl1, Wr1, g1, b1)
    p2 = _sc_agg(False)(h1, packed2, zrow)
    out = _tc_layer(p2, cnt, h1, Wl2, bl2, Wr2, g2, b2)
    return out


# speedup vs baseline: 1.5177x; 1.2031x over previous
import functools

import jax
import jax.numpy as jnp
from jax import lax
from jax.experimental import pallas as pl
from jax.experimental.pallas import tpu as pltpu
from jax.experimental.pallas import tpu_sc as plsc

N = 10000
E = 320000
D = 128
EPS = 1e-5

NC = 2
NS = 16
NW = NC * NS
EPW = E // NW
CHUNK = 80
NCHUNK = EPW // CHUNK
RPT = 632
N_PAD = NS * RPT
CW = 128


def _mesh():
    return plsc.VectorSubcoreMesh(core_axis_name="c", subcore_axis_name="s",
                                  num_cores=NC, num_subcores=NS)


def _sc_agg_body(with_count, *refs):
    if with_count:
        (table, packed3, zrow, ones_h, out, cnt_out, packed_all,
         sa_a, da_a, sa_b, da_b, rows_a, rows_b, acc, sem_a, sem_b) = refs
    else:
        (table, packed3, zrow, out, packed_all,
         sa_a, da_a, sa_b, da_b, rows_a, rows_b, acc, sem_a, sem_b) = refs

    c = lax.axis_index("c")
    s = lax.axis_index("s")
    wid = s * NC + c

    r0 = s * RPT
    pltpu.sync_copy(zrow.at[pl.ds(r0, RPT)], acc.at[pl.ds(r0, RPT)])

    pltpu.sync_copy(packed3.at[wid], packed_all)

    def unpack(i, sa, da):
        for k in range(CHUNK // 16):
            w = packed_all[i, pl.ds(k * 16, 16)]
            sa[pl.ds(k * 16, 16)] = w & 0xFFFF
            da[pl.ds(k * 16, 16)] = jnp.right_shift(w, 16)

    def wait_gather(sa, buf, sem):
        pltpu.make_async_copy(table.at[sa], buf, sem).wait()

    if with_count:
        pltpu.sync_copy(ones_h, rows_a)
        plsc.subcore_barrier()

        def cstep(i, carry):
            unpack(i, sa_a, da_a)
            pltpu.sync_copy(rows_a, acc.at[da_a], add=True)
            return carry

        lax.fori_loop(0, NCHUNK, cstep, 0)
        plsc.subcore_barrier()
        pltpu.sync_copy(acc.at[pl.ds(r0, RPT)], cnt_out.at[c, pl.ds(r0, RPT)])
        pltpu.sync_copy(zrow.at[pl.ds(r0, RPT)], acc.at[pl.ds(r0, RPT)])

    plsc.subcore_barrier()

    unpack(0, sa_a, da_a)
    pltpu.async_copy(table.at[sa_a], rows_a, sem_a)

    def pair(j, carry):
        i0 = 2 * j
        unpack(i0 + 1, sa_b, da_b)
        pltpu.async_copy(table.at[sa_b], rows_b, sem_b)
        wait_gather(sa_a, rows_a, sem_a)
        pltpu.sync_copy(rows_a, acc.at[da_a], add=True)
        unpack(i0 + 2, sa_a, da_a)
        pltpu.async_copy(table.at[sa_a], rows_a, sem_a)
        wait_gather(sa_b, rows_b, sem_b)
        pltpu.sync_copy(rows_b, acc.at[da_b], add=True)
        return carry

    lax.fori_loop(0, (NCHUNK - 1) // 2, pair, 0)
    wait_gather(sa_a, rows_a, sem_a)
    pltpu.sync_copy(rows_a, acc.at[da_a], add=True)

    plsc.subcore_barrier()

    pltpu.sync_copy(acc.at[pl.ds(r0, RPT)], out.at[c, pl.ds(r0, RPT)])


@functools.lru_cache(maxsize=None)
def _sc_agg(with_count):
    if with_count:
        out_type = (jax.ShapeDtypeStruct((NC, N_PAD, D), jnp.float32),
                    jax.ShapeDtypeStruct((NC, N_PAD, CW), jnp.float32))
    else:
        out_type = jax.ShapeDtypeStruct((NC, N_PAD, D), jnp.float32)
    return pl.kernel(
        functools.partial(_sc_agg_body, with_count),
        out_type=out_type,
        mesh=_mesh(),
        scratch_types=(
            pltpu.VMEM((NCHUNK, CHUNK), jnp.int32),
            pltpu.VMEM((CHUNK,), jnp.int32),
            pltpu.VMEM((CHUNK,), jnp.int32),
            pltpu.VMEM((CHUNK,), jnp.int32),
            pltpu.VMEM((CHUNK,), jnp.int32),
            pltpu.VMEM((CHUNK, D), jnp.float32),
            pltpu.VMEM((CHUNK, D), jnp.float32),
            pltpu.VMEM_SHARED((N_PAD, D), jnp.float32),
            pltpu.SemaphoreType.DMA,
            pltpu.SemaphoreType.DMA,
        ))


def _relu_body(x_ref, o_ref):
    o_ref[...] = jnp.maximum(x_ref[...], 0.0)


def _tc_relu(x):
    return pl.pallas_call(
        _relu_body,
        out_shape=jax.ShapeDtypeStruct((N, D), jnp.float32),
    )(x)


def _tc_layer_body(p_ref, cnt_ref, xin_ref, wl_ref, bl_ref, wr_ref,
                   g_ref, b_ref, out_ref):
    ssum = p_ref[0, :N] + p_ref[1, :N]
    cn = cnt_ref[0, :N] + cnt_ref[1, :N]
    inv = 1.0 / jnp.maximum(cn[:, 0:1], 1.0)
    agg = ssum * inv
    t = (jnp.dot(agg, wl_ref[...], preferred_element_type=jnp.float32)
         + jnp.dot(xin_ref[...], wr_ref[...], preferred_element_type=jnp.float32)
         + bl_ref[...][None, :])
    mean = jnp.mean(t, axis=0)
    var = jnp.mean((t - mean[None, :]) ** 2, axis=0)
    h = (g_ref[...][None, :] * (t - mean[None, :]) * lax.rsqrt(var + EPS)
         + b_ref[...][None, :])
    out_ref[...] = jnp.maximum(h, 0.0)


def _tc_layer(p, cnt, xin, wl, bl, wr, g, b):
    return pl.pallas_call(
        _tc_layer_body,
        out_shape=jax.ShapeDtypeStruct((N, D), jnp.float32),
    )(p, cnt, xin, wl, bl, wr, g, b)


def kernel(x, edge_index, Wl1, bl1, Wr1, g1, b1, Wl2, bl2, Wr2, g2, b2):
    src = edge_index[0].astype(jnp.int32)
    dst = edge_index[1].astype(jnp.int32)
    packed3 = (src | (dst << 16)).reshape(NW, NCHUNK, CHUNK)
    zrow = jnp.zeros((N_PAD, D), jnp.float32)
    ones_h = jnp.ones((CHUNK, CW), jnp.float32)

    r1 = _tc_relu(x)
    p1, cnt = _sc_agg(True)(r1, packed3, zrow, ones_h)
    h1 = _tc_layer(p1, cnt, x, Wl1, bl1, Wr1, g1, b1)
    p2 = _sc_agg(False)(h1, packed3, zrow)
    out = _tc_layer(p2, cnt, h1, Wl2, bl2, Wr2, g2, b2)
    return out
